# baseline (speedup 1.0000x reference)
import jax
import jax.numpy as jnp
from jax.experimental import pallas as pl
from jax.experimental.pallas import tpu as pltpu

ALPHA = 0.1


def _prologue_kernel(x_ref, w0t_ref, b0_ref, cw0_ref, h0_ref, xx1_ref):
    h0 = jnp.dot(x_ref[...], w0t_ref[...],
                 preferred_element_type=jnp.float32) + b0_ref[...]
    h0_ref[...] = h0
    xx1_ref[...] = jnp.dot(jax.nn.relu(h0), cw0_ref[...],
                           preferred_element_type=jnp.float32)


def _layer_kernel(adj_ref, xx_ref, h0_ref, bpre_ref, wep_ref, bep_ref, out_ref):
    hi = jnp.dot(adj_ref[...], xx_ref[...],
                 preferred_element_type=jnp.float32)
    t = jax.nn.relu((1.0 - ALPHA) * hi + ALPHA * h0_ref[...] + bpre_ref[...])
    out_ref[...] = jnp.dot(t, wep_ref[...],
                           preferred_element_type=jnp.float32) + bep_ref[...]


def _layer_call(adj, xx, h0, bpre, wep, bep, block_rows):
    n, k = adj.shape
    f = xx.shape[1]
    co = wep.shape[1]
    return pl.pallas_call(
        _layer_kernel,
        grid=(n // block_rows,),
        in_specs=[
            pl.BlockSpec((block_rows, k), lambda i: (i, 0)),
            pl.BlockSpec((k, f), lambda i: (0, 0)),
            pl.BlockSpec((block_rows, f), lambda i: (i, 0)),
            pl.BlockSpec((1, f), lambda i: (0, 0)),
            pl.BlockSpec((f, co), lambda i: (0, 0)),
            pl.BlockSpec((1, co), lambda i: (0, 0)),
        ],
        out_specs=pl.BlockSpec((block_rows, co), lambda i: (i, 0)),
        out_shape=jax.ShapeDtypeStruct((n, co), jnp.float32),
        compiler_params=pltpu.CompilerParams(
            dimension_semantics=("parallel",),
        ),
    )(adj, xx, h0, bpre, wep, bep)


def kernel(x, adj, fc0_w, fc0_b, conv_w, conv_b, fc1_w, fc1_b):
    n, nfeat = x.shape
    nhid = fc0_w.shape[0]
    nclass = fc1_w.shape[0]

    h0, xx1 = pl.pallas_call(
        _prologue_kernel,
        out_shape=(
            jax.ShapeDtypeStruct((n, nhid), jnp.float32),
            jax.ShapeDtypeStruct((n, nhid), jnp.float32),
        ),
    )(x, fc0_w.T, fc0_b.reshape(1, nhid), conv_w[0])

    block_rows = 200
    xx2 = _layer_call(adj, xx1, h0, conv_b[0], conv_w[1],
                      jnp.zeros((1, nhid), jnp.float32), block_rows)
    out = _layer_call(adj, xx2, h0, conv_b[1], fc1_w.T,
                      fc1_b.reshape(1, nclass), block_rows)
    return out


# speedup vs baseline: 1.0966x; 1.0966x over previous
import jax
import jax.numpy as jnp
from jax.experimental import pallas as pl
from jax.experimental.pallas import tpu as pltpu

ALPHA = 0.1


def _fused_kernel(x_ref, adj_ref, w0t_ref, b0_ref, cw0_ref, cb_ref,
                  cw1_ref, w1t_ref, b1_ref, out_ref,
                  h0_ref, xx1_ref, xx2_ref):
    l = pl.program_id(0)
    i = pl.program_id(1)
    r = adj_ref.shape[0]
    base = i * r

    @pl.when(jnp.logical_and(l == 0, i == 0))
    def _prologue():
        h0 = jnp.dot(x_ref[...], w0t_ref[...],
                     preferred_element_type=jnp.float32) + b0_ref[...]
        h0_ref[...] = h0
        xx1_ref[...] = jnp.dot(jax.nn.relu(h0), cw0_ref[...],
                               preferred_element_type=jnp.float32)

    h0_blk = h0_ref[pl.ds(base, r), :]

    @pl.when(l == 0)
    def _layer0():
        hi = jnp.dot(adj_ref[...], xx1_ref[...],
                     preferred_element_type=jnp.float32)
        t = jax.nn.relu((1.0 - ALPHA) * hi + ALPHA * h0_blk + cb_ref[0])
        xx2_ref[pl.ds(base, r), :] = jnp.dot(
            t, cw1_ref[...], preferred_element_type=jnp.float32)

    @pl.when(l == 1)
    def _layer1():
        hi = jnp.dot(adj_ref[...], xx2_ref[...],
                     preferred_element_type=jnp.float32)
        t = jax.nn.relu((1.0 - ALPHA) * hi + ALPHA * h0_blk + cb_ref[0])
        out_ref[...] = jnp.dot(t, w1t_ref[...],
                               preferred_element_type=jnp.float32) + b1_ref[...]


def kernel(x, adj, fc0_w, fc0_b, conv_w, conv_b, fc1_w, fc1_b):
    n, nfeat = x.shape
    nhid = fc0_w.shape[0]
    nclass = fc1_w.shape[0]
    block_rows = 400
    nblk = n // block_rows

    return pl.pallas_call(
        _fused_kernel,
        grid=(2, nblk),
        in_specs=[
            pl.BlockSpec((n, nfeat), lambda l, i: (0, 0)),
            pl.BlockSpec((block_rows, n), lambda l, i: (i, 0)),
            pl.BlockSpec((nfeat, nhid), lambda l, i: (0, 0)),
            pl.BlockSpec((1, nhid), lambda l, i: (0, 0)),
            pl.BlockSpec((nhid, nhid), lambda l, i: (0, 0)),
            pl.BlockSpec((1, 1, nhid), lambda l, i: (l, 0, 0)),
            pl.BlockSpec((nhid, nhid), lambda l, i: (0, 0)),
            pl.BlockSpec((nhid, nclass), lambda l, i: (0, 0)),
            pl.BlockSpec((1, nclass), lambda l, i: (0, 0)),
        ],
        out_specs=pl.BlockSpec((block_rows, nclass), lambda l, i: (i, 0)),
        out_shape=jax.ShapeDtypeStruct((n, nclass), jnp.float32),
        scratch_shapes=[
            pltpu.VMEM((n, nhid), jnp.float32),
            pltpu.VMEM((n, nhid), jnp.float32),
            pltpu.VMEM((n, nhid), jnp.float32),
        ],
        compiler_params=pltpu.CompilerParams(
            dimension_semantics=("arbitrary", "arbitrary"),
        ),
    )(x, adj, fc0_w.T, fc0_b.reshape(1, nhid), conv_w[0], conv_b,
      conv_w[1], fc1_w.T, fc1_b.reshape(1, nclass))
